# baseline (speedup 1.0000x reference)
import functools

import jax
import jax.numpy as jnp
from jax import lax
from jax.experimental import pallas as pl
from jax.experimental.pallas import tpu as pltpu
from jax.experimental.pallas import tpu_sc as plsc

NUM_EMB = 1000000
DIM = 64
BATCH = 16384
SEQ = 50
B = BATCH * SEQ
NW = 32
NPIECE = 4
PIECE = B // NPIECE
BPW = PIECE // NW
CHUNK = 640
NPAIR = BPW // (2 * CHUNK)

_mesh = plsc.VectorSubcoreMesh(core_axis_name="c", subcore_axis_name="s")


@functools.partial(
    pl.kernel,
    mesh=_mesh,
    out_type=jax.ShapeDtypeStruct((PIECE, DIM), jnp.float32),
    scratch_types=[
        pltpu.VMEM((CHUNK,), jnp.int32),
        pltpu.VMEM((CHUNK,), jnp.int32),
        pltpu.VMEM((CHUNK, DIM), jnp.float32),
        pltpu.VMEM((CHUNK, DIM), jnp.float32),
        pltpu.SemaphoreType.DMA,
        pltpu.SemaphoreType.DMA,
    ],
    compiler_params=pltpu.CompilerParams(use_tc_tiling_on_sc=False),
)
def _embed_gather(table_hbm, idx_hbm, out_hbm,
                  idx_v0, idx_v1, rows_v0, rows_v1, sem0, sem1):
    wid = lax.axis_index("s") * 2 + lax.axis_index("c")
    base = wid * BPW

    def fetch(off, idx_v, rows_v, sem):
        pltpu.sync_copy(idx_hbm.at[pl.ds(off, CHUNK)], idx_v)
        return pltpu.async_copy(table_hbm.at[idx_v], rows_v, sem)

    def drain(off, rows_v, copy):
        copy.wait()
        pltpu.sync_copy(rows_v, out_hbm.at[pl.ds(off, CHUNK)])

    def body(i, carry):
        off0 = base + i * (2 * CHUNK)
        off1 = off0 + CHUNK
        c0 = fetch(off0, idx_v0, rows_v0, sem0)
        c1 = fetch(off1, idx_v1, rows_v1, sem1)
        drain(off0, rows_v0, c0)
        drain(off1, rows_v1, c1)
        return carry

    lax.fori_loop(0, NPAIR, body, 0)


def kernel(token_ids, embed_matrix):
    flat = token_ids.reshape(-1).astype(jnp.int32)
    pieces = [
        _embed_gather(embed_matrix, flat[p * PIECE:(p + 1) * PIECE])
        for p in range(NPIECE)
    ]
    out = jnp.concatenate(pieces, axis=0)
    return out.reshape(token_ids.shape + (DIM,))


# speedup vs baseline: 1.5280x; 1.5280x over previous
import functools

import jax
import jax.numpy as jnp
from jax import lax
from jax.experimental import pallas as pl
from jax.experimental.pallas import tpu as pltpu
from jax.experimental.pallas import tpu_sc as plsc

NUM_EMB = 1000000
DIM = 64
BATCH = 16384
SEQ = 50
B = BATCH * SEQ
NW = 32
BPW = B // NW
CHUNK = 512
NPAIR = BPW // (2 * CHUNK)

_mesh = plsc.VectorSubcoreMesh(core_axis_name="c", subcore_axis_name="s")


@functools.partial(
    pl.kernel,
    mesh=_mesh,
    out_type=jax.ShapeDtypeStruct((B, DIM), jnp.float32),
    scratch_types=[
        pltpu.VMEM((CHUNK,), jnp.int32),
        pltpu.VMEM((CHUNK,), jnp.int32),
        pltpu.VMEM((CHUNK, DIM), jnp.float32),
        pltpu.VMEM((CHUNK, DIM), jnp.float32),
        pltpu.SemaphoreType.DMA,
        pltpu.SemaphoreType.DMA,
    ],
    compiler_params=pltpu.CompilerParams(use_tc_tiling_on_sc=False),
)
def _embed_gather(table_hbm, idx_hbm, out_hbm,
                  idx_v0, idx_v1, rows_v0, rows_v1, sem0, sem1):
    wid = lax.axis_index("s") * 2 + lax.axis_index("c")
    base = wid * BPW

    def fetch(off, idx_v, rows_v, sem):
        pltpu.sync_copy(idx_hbm.at[pl.ds(off, CHUNK)], idx_v)
        return pltpu.async_copy(table_hbm.at[idx_v], rows_v, sem)

    def drain(off, rows_v, copy):
        copy.wait()
        pltpu.sync_copy(rows_v, out_hbm.at[pl.ds(off, CHUNK)])

    def body(i, carry):
        off0 = base + i * (2 * CHUNK)
        off1 = off0 + CHUNK
        c0 = fetch(off0, idx_v0, rows_v0, sem0)
        c1 = fetch(off1, idx_v1, rows_v1, sem1)
        drain(off0, rows_v0, c0)
        drain(off1, rows_v1, c1)
        return carry

    lax.fori_loop(0, NPAIR, body, 0)


def kernel(token_ids, embed_matrix):
    flat = token_ids.reshape(-1).astype(jnp.int32)
    out = _embed_gather(embed_matrix, flat)
    return out.reshape(token_ids.shape + (DIM,))


# speedup vs baseline: 1.5343x; 1.0041x over previous
import functools

import jax
import jax.numpy as jnp
from jax import lax
from jax.experimental import pallas as pl
from jax.experimental.pallas import tpu as pltpu
from jax.experimental.pallas import tpu_sc as plsc

NUM_EMB = 1000000
DIM = 64
BATCH = 16384
SEQ = 50
B = BATCH * SEQ
NW = 32
BPW = B // NW
CHUNK = 800
NPAIR = BPW // (2 * CHUNK)

_mesh = plsc.VectorSubcoreMesh(core_axis_name="c", subcore_axis_name="s")


@functools.partial(
    pl.kernel,
    mesh=_mesh,
    out_type=jax.ShapeDtypeStruct((B, DIM), jnp.float32),
    scratch_types=[
        pltpu.VMEM((CHUNK,), jnp.int32),
        pltpu.VMEM((CHUNK,), jnp.int32),
        pltpu.VMEM((CHUNK, DIM), jnp.float32),
        pltpu.VMEM((CHUNK, DIM), jnp.float32),
        pltpu.SemaphoreType.DMA,
        pltpu.SemaphoreType.DMA,
    ],
    compiler_params=pltpu.CompilerParams(use_tc_tiling_on_sc=False),
)
def _embed_gather(table_hbm, idx_hbm, out_hbm,
                  idx_v0, idx_v1, rows_v0, rows_v1, sem0, sem1):
    wid = lax.axis_index("s") * 2 + lax.axis_index("c")
    base = wid * BPW

    def fetch(off, idx_v, rows_v, sem):
        pltpu.sync_copy(idx_hbm.at[pl.ds(off, CHUNK)], idx_v)
        return pltpu.async_copy(table_hbm.at[idx_v], rows_v, sem)

    def drain(off, rows_v, copy):
        copy.wait()
        pltpu.sync_copy(rows_v, out_hbm.at[pl.ds(off, CHUNK)])

    def body(i, carry):
        off0 = base + i * (2 * CHUNK)
        off1 = off0 + CHUNK
        c0 = fetch(off0, idx_v0, rows_v0, sem0)
        c1 = fetch(off1, idx_v1, rows_v1, sem1)
        drain(off0, rows_v0, c0)
        drain(off1, rows_v1, c1)
        return carry

    lax.fori_loop(0, NPAIR, body, 0)


def kernel(token_ids, embed_matrix):
    flat = token_ids.reshape(-1).astype(jnp.int32)
    out = _embed_gather(embed_matrix, flat)
    return out.reshape(token_ids.shape + (DIM,))
